# baseline (speedup 1.0000x reference)
import functools

import jax
import jax.numpy as jnp
from jax import lax
from jax.experimental import pallas as pl
from jax.experimental.pallas import tpu as pltpu
from jax.experimental.pallas import tpu_sc as plsc

EPS_BN = 1e-5
EPS_LN = 1e-5

try:
    _info = plsc.get_sparse_core_info()
    _NC, _NS = _info.num_cores, _info.num_subcores
except Exception:
    _NC, _NS = 2, 16
_NW = _NC * _NS

_CHUNK = 128


def _chunks(total, step):
    out = []
    o = 0
    while o < total:
        out.append((o, min(step, total - o)))
        o += step
    return out


def _rows_per_subcore(N):
    return ((N + _NS * 8 - 1) // (_NS * 8)) * 8


@functools.lru_cache(maxsize=None)
def _make_sc_spmm(N, E, D, gather=True):
    nch = E // _CHUNK
    assert E % _CHUNK == 0 and nch % (8 * _NW) == 0
    nw_f = nch // _NW
    rps = _rows_per_subcore(N)
    NP = rps * _NS
    zc = _chunks(rps, _CHUNK)
    budget = (2097151 - NP * D) // _NS
    PH = nw_f
    while 2 * PH * _CHUNK + 2 * _CHUNK * D > budget:
        PH = (PH + 1) // 2
    while nw_f % PH:
        PH -= 1
    nphase = nw_f // PH
    pairs, odd = divmod(PH, 2)
    mesh = plsc.VectorSubcoreMesh(core_axis_name="c", subcore_axis_name="s")

    @functools.partial(
        pl.kernel,
        out_type=jax.ShapeDtypeStruct((_NC, NP, D), jnp.float32),
        mesh=mesh,
        scratch_types=[
            pltpu.VMEM((PH, _CHUNK), jnp.int32),
            pltpu.VMEM((PH, _CHUNK), jnp.int32),
            pltpu.VMEM((_CHUNK, D), jnp.float32),
            pltpu.VMEM((_CHUNK, D), jnp.float32),
            pltpu.VMEM_SHARED((NP, D), jnp.float32),
            pltpu.SemaphoreType.DMA,
            pltpu.SemaphoreType.DMA,
        ],
    )
    def sc_spmm(rows_hbm, src_hbm, dst_hbm, zeros_hbm, out_hbm, sidx, didx,
                buf_a, buf_b, acc_sh, sem_a, sem_b):
        cid = lax.axis_index("c")
        sid = lax.axis_index("s")
        wid = sid * _NC + cid
        r0 = sid * rps
        pltpu.sync_copy(zeros_hbm, buf_a)
        for o, sz in zc:
            pltpu.sync_copy(buf_a.at[pl.ds(0, sz)],
                            acc_sh.at[pl.ds(r0 + o, sz)])
        cbase = wid * nw_f
        if not gather:
            pltpu.sync_copy(rows_hbm, buf_a)
        plsc.subcore_barrier()

        if gather:

            def phase(p, carry):
                pb = cbase + p * PH
                pltpu.sync_copy(src_hbm.at[pl.ds(pb, PH)], sidx)
                pltpu.sync_copy(dst_hbm.at[pl.ds(pb, PH)], didx)
                pltpu.async_copy(rows_hbm.at[sidx.at[0]], buf_a, sem_a)

                def body(i, carry):
                    c0 = 2 * i
                    pltpu.async_copy(rows_hbm.at[sidx.at[c0 + 1]], buf_b,
                                     sem_b)
                    pltpu.make_async_copy(rows_hbm.at[sidx.at[c0]], buf_a,
                                          sem_a).wait()
                    pltpu.sync_copy(buf_a, acc_sh.at[didx.at[c0]], add=True)
                    cn = jnp.minimum(c0 + 2, PH - 1)
                    pltpu.async_copy(rows_hbm.at[sidx.at[cn]], buf_a, sem_a)
                    pltpu.make_async_copy(rows_hbm.at[sidx.at[c0 + 1]],
                                          buf_b, sem_b).wait()
                    pltpu.sync_copy(buf_b, acc_sh.at[didx.at[c0 + 1]],
                                    add=True)
                    return carry

                lax.fori_loop(0, pairs, body, 0)
                pltpu.make_async_copy(rows_hbm.at[sidx.at[PH - 1]], buf_a,
                                      sem_a).wait()
                if odd:
                    pltpu.sync_copy(buf_a, acc_sh.at[didx.at[PH - 1]],
                                    add=True)
                return carry

            lax.fori_loop(0, nphase, phase, 0)
        else:

            def phase(p, carry):
                pb = cbase + p * PH
                pltpu.sync_copy(dst_hbm.at[pl.ds(pb, PH)], didx)

                def body(i, carry):
                    pltpu.sync_copy(buf_a, acc_sh.at[didx.at[i]], add=True)
                    return carry

                lax.fori_loop(0, PH, body, 0)
                return carry

            lax.fori_loop(0, nphase, phase, 0)

        plsc.subcore_barrier()
        for o, sz in zc:
            pltpu.sync_copy(acc_sh.at[pl.ds(r0 + o, sz)],
                            buf_a.at[pl.ds(0, sz)])
            pltpu.sync_copy(buf_a.at[pl.ds(0, sz)],
                            out_hbm.at[cid, pl.ds(r0 + o, sz)])

    return sc_spmm


_BN_S = 1.0 / (1.0 + EPS_BN) ** 0.5


def _ln_body(a, w, b):
    mu = jnp.mean(a, axis=-1, keepdims=True)
    var = jnp.mean((a - mu) ** 2, axis=-1, keepdims=True)
    return (a - mu) * lax.rsqrt(var + EPS_LN) * w[None, :] + b[None, :]


def _tc_init_body(x_ref, gfw_ref, gfb_ref, gbw_ref, gbb_ref, tfw_ref, tfb_ref,
                  tlw_ref, tlb_ref, h0_ref, z0_ref):
    x = x_ref[...]
    h = jnp.dot(x, gfw_ref[...], preferred_element_type=jnp.float32)
    h = h + gfb_ref[...][None, :]
    h = h * (_BN_S * gbw_ref[...])[None, :] + gbb_ref[...][None, :]
    h0_ref[...] = jnp.maximum(h, 0.0)
    z = jnp.dot(x, tfw_ref[...], preferred_element_type=jnp.float32)
    z = z + tfb_ref[...][None, :]
    z = _ln_body(z, tlw_ref[...], tlb_ref[...])
    z0_ref[...] = jnp.maximum(z, 0.0)


def _tc_trans_body(z0_ref, qkv_ref, lnw_ref, lnb_ref, x1_ref, *, layers, n):
    z = z0_ref[...]
    last = z
    fn = jnp.float32(n)
    for l in range(layers):
        q = jnp.dot(z, qkv_ref[l, 0], preferred_element_type=jnp.float32)
        k = jnp.dot(z, qkv_ref[l, 1], preferred_element_type=jnp.float32)
        v = jnp.dot(z, qkv_ref[l, 2], preferred_element_type=jnp.float32)
        inv_qk = lax.rsqrt(jnp.sum(q * q) * jnp.sum(k * k))
        kvs = lax.dot_general(k, v, (((0,), (0,)), ((), ())),
                              preferred_element_type=jnp.float32)
        ks = jnp.sum(k, axis=0)
        num = jnp.dot(q, kvs, preferred_element_type=jnp.float32) * inv_qk \
            + fn * v
        den = jnp.sum(q * ks[None, :], axis=1, keepdims=True) * inv_qk + fn
        a = (num / den + last) * 0.5
        a = _ln_body(a, lnw_ref[l + 1], lnb_ref[l + 1])
        z = jnp.maximum(a, 0.0)
        last = z
    x1_ref[...] = z


def _tc_dinv_hw_body(degp_ref, h0_ref, w1_ref, dinv_ref, hws_ref, *, D):
    n = h0_ref.shape[0]
    d = degp_ref[0, 0:n, 0:1] + degp_ref[1, 0:n, 0:1] + 1.0
    dinv = jnp.broadcast_to(lax.rsqrt(d), (n, D))
    dinv_ref[...] = dinv
    hw = jnp.dot(h0_ref[...], w1_ref[...], preferred_element_type=jnp.float32)
    hws_ref[...] = dinv * hw


def _tc_gcn_body(p_ref, hws_ref, h_ref, dinv_ref, bnw_ref, bnb_ref, cb_ref,
                 wn_ref, hn_ref, hwsn_ref):
    dinv = dinv_ref[...]
    n = dinv.shape[0]
    agg = dinv * (p_ref[0, 0:n] + p_ref[1, 0:n] + hws_ref[...]) \
        + cb_ref[...][None, :]
    c = jnp.maximum(agg * (_BN_S * bnw_ref[...])[None, :]
                    + bnb_ref[...][None, :], 0.0)
    hn = c + h_ref[...]
    hn_ref[...] = hn
    if wn_ref is not None:
        hw = jnp.dot(hn, wn_ref[...], preferred_element_type=jnp.float32)
        hwsn_ref[...] = dinv * hw


def _tc_gcn_last_body(p_ref, hws_ref, h_ref, dinv_ref, bnw_ref, bnb_ref,
                      cb_ref, hn_ref):
    _tc_gcn_body(p_ref, hws_ref, h_ref, dinv_ref, bnw_ref, bnb_ref, cb_ref,
                 None, hn_ref, None)


def _tc_final_body(h_ref, x1_ref, fcw_ref, fcb_ref, out_ref):
    o = 0.5 * h_ref[...] + 0.5 * x1_ref[...]
    t = jnp.dot(o, fcw_ref[...], preferred_element_type=jnp.float32)
    t = t + fcb_ref[...][None, :]
    m = jnp.max(t, axis=-1, keepdims=True)
    e = jnp.exp(t - m)
    s = jnp.sum(e, axis=-1, keepdims=True)
    out_ref[...] = t - m - jnp.log(s)


def _tc_call(body, out_shapes, *args, **static):
    if static:
        body = functools.partial(body, **static)
    return pl.pallas_call(body, out_shape=out_shapes)(*args)


def kernel(x, edge_index, batch, g_fc_w, g_fc_b, g_bn_w, g_bn_b, g_conv_w,
           g_conv_b, t_fc_w, t_fc_b, t_ln_w, t_ln_b, t_qkv_w, fc_w, fc_b):
    N, D_IN = x.shape
    E = edge_index.shape[1]
    HID = g_fc_w.shape[1]
    OUT = fc_w.shape[1]
    gnn_layers = g_conv_w.shape[0]
    trans_layers = t_qkv_w.shape[0]

    rps = _rows_per_subcore(N)
    NP = rps * _NS
    align = _CHUNK * 8 * _NW
    E_p = -(-E // align) * align
    npad = E_p - E
    if npad:
        pad_src = jnp.zeros((npad,), jnp.int32)
        pad_dst = N + (jnp.arange(npad, dtype=jnp.int32) % (NP - N))
        src_f = jnp.concatenate([edge_index[0], pad_src])
        dst_f = jnp.concatenate([edge_index[1], pad_dst])
    else:
        src_f = edge_index[0]
        dst_f = edge_index[1]
    src2 = src_f.reshape(E_p // _CHUNK, _CHUNK)
    dst2 = dst_f.reshape(E_p // _CHUNK, _CHUNK)
    f32 = jnp.float32

    zeros_d = jnp.zeros((_CHUNK, HID), f32)
    ones_d = jnp.ones((_CHUNK, HID), f32)

    spmm = _make_sc_spmm(N, E_p, HID)

    degp = _make_sc_spmm(N, E_p, HID, gather=False)(ones_d, src2, dst2, zeros_d)

    h0, z0 = _tc_call(
        _tc_init_body,
        (jax.ShapeDtypeStruct((N, HID), f32),) * 2,
        x, g_fc_w, g_fc_b, g_bn_w[0], g_bn_b[0], t_fc_w, t_fc_b,
        t_ln_w[0], t_ln_b[0])

    x1 = _tc_call(
        _tc_trans_body,
        jax.ShapeDtypeStruct((N, HID), f32),
        z0, t_qkv_w, t_ln_w, t_ln_b, layers=trans_layers, n=N)

    dinv, hws = _tc_call(
        _tc_dinv_hw_body,
        (jax.ShapeDtypeStruct((N, HID), f32),) * 2,
        degp, h0, g_conv_w[0], D=HID)

    h = h0
    for l in range(gnn_layers):
        p = spmm(hws, src2, dst2, zeros_d)
        if l + 1 < gnn_layers:
            h, hws = _tc_call(
                _tc_gcn_body,
                (jax.ShapeDtypeStruct((N, HID), f32),) * 2,
                p, hws, h, dinv, g_bn_w[l + 1], g_bn_b[l + 1], g_conv_b[l],
                g_conv_w[l + 1])
        else:
            h = _tc_call(
                _tc_gcn_last_body,
                jax.ShapeDtypeStruct((N, HID), f32),
                p, hws, h, dinv, g_bn_w[l + 1], g_bn_b[l + 1], g_conv_b[l])

    return _tc_call(
        _tc_final_body,
        jax.ShapeDtypeStruct((N, OUT), f32),
        h, x1, fc_w, fc_b)


# speedup vs baseline: 3.3889x; 3.3889x over previous
import functools

import jax
import jax.numpy as jnp
from jax import lax
from jax.experimental import pallas as pl
from jax.experimental.pallas import tpu as pltpu
from jax.experimental.pallas import tpu_sc as plsc

EPS_BN = 1e-5
EPS_LN = 1e-5

try:
    _info = plsc.get_sparse_core_info()
    _NC, _NS = _info.num_cores, _info.num_subcores
except Exception:
    _NC, _NS = 2, 16
_NW = _NC * _NS

_CHUNK = 128


def _chunks(total, step):
    out = []
    o = 0
    while o < total:
        out.append((o, min(step, total - o)))
        o += step
    return out


def _rows_per_subcore(N):
    return ((N + _NS * 8 - 1) // (_NS * 8)) * 8


@functools.lru_cache(maxsize=None)
def _make_sc_spmm(N, E, D, gather=True):
    nch = E // _CHUNK
    assert E % _CHUNK == 0 and nch % (8 * _NW) == 0
    nw_f = nch // _NW
    rps = _rows_per_subcore(N)
    NP = rps * _NS
    zc = _chunks(rps, _CHUNK)
    budget = (2097151 - NP * D) // _NS
    PH = nw_f
    while 2 * PH * _CHUNK + 2 * _CHUNK * D > budget:
        PH = (PH + 1) // 2
    while nw_f % PH:
        PH -= 1
    nphase = nw_f // PH
    pairs, odd = divmod(PH, 2)
    mesh = plsc.VectorSubcoreMesh(core_axis_name="c", subcore_axis_name="s")

    @functools.partial(
        pl.kernel,
        out_type=jax.ShapeDtypeStruct((_NC, NP, D), jnp.float32),
        mesh=mesh,
        scratch_types=[
            pltpu.VMEM((PH, _CHUNK), jnp.int32),
            pltpu.VMEM((PH, _CHUNK), jnp.int32),
            pltpu.VMEM((_CHUNK, D), jnp.float32),
            pltpu.VMEM((_CHUNK, D), jnp.float32),
            pltpu.VMEM_SHARED((NP, D), jnp.float32),
            pltpu.SemaphoreType.DMA,
            pltpu.SemaphoreType.DMA,
        ],
    )
    def sc_spmm(rows_hbm, src_hbm, dst_hbm, zeros_hbm, out_hbm, sidx, didx,
                buf_a, buf_b, acc_sh, sem_a, sem_b):
        cid = lax.axis_index("c")
        sid = lax.axis_index("s")
        wid = sid * _NC + cid
        r0 = sid * rps
        pltpu.sync_copy(zeros_hbm, buf_a)
        for o, sz in zc:
            pltpu.sync_copy(buf_a.at[pl.ds(0, sz)],
                            acc_sh.at[pl.ds(r0 + o, sz)])
        cbase = wid * nw_f
        if not gather:
            pltpu.sync_copy(rows_hbm, buf_a)
        plsc.subcore_barrier()

        if gather:

            def phase(p, carry):
                pb = cbase + p * PH
                pltpu.sync_copy(src_hbm.at[pl.ds(pb, PH)], sidx)
                pltpu.sync_copy(dst_hbm.at[pl.ds(pb, PH)], didx)
                pltpu.async_copy(rows_hbm.at[sidx.at[0]], buf_a, sem_a)

                def body(i, carry):
                    c0 = 2 * i
                    pltpu.async_copy(rows_hbm.at[sidx.at[c0 + 1]], buf_b,
                                     sem_b)
                    pltpu.make_async_copy(rows_hbm.at[sidx.at[c0]], buf_a,
                                          sem_a).wait()
                    pltpu.sync_copy(buf_a, acc_sh.at[didx.at[c0]], add=True)
                    cn = jnp.minimum(c0 + 2, PH - 1)
                    pltpu.async_copy(rows_hbm.at[sidx.at[cn]], buf_a, sem_a)
                    pltpu.make_async_copy(rows_hbm.at[sidx.at[c0 + 1]],
                                          buf_b, sem_b).wait()
                    pltpu.sync_copy(buf_b, acc_sh.at[didx.at[c0 + 1]],
                                    add=True)
                    return carry

                lax.fori_loop(0, pairs, body, 0)
                pltpu.make_async_copy(rows_hbm.at[sidx.at[PH - 1]], buf_a,
                                      sem_a).wait()
                if odd:
                    pltpu.sync_copy(buf_a, acc_sh.at[didx.at[PH - 1]],
                                    add=True)
                return carry

            lax.fori_loop(0, nphase, phase, 0)
        else:

            def phase(p, carry):
                pb = cbase + p * PH
                pltpu.sync_copy(dst_hbm.at[pl.ds(pb, PH)], didx)

                def body(i, carry):
                    pltpu.sync_copy(buf_a, acc_sh.at[didx.at[i]], add=True)
                    return carry

                lax.fori_loop(0, PH, body, 0)
                return carry

            lax.fori_loop(0, nphase, phase, 0)

        plsc.subcore_barrier()
        for o, sz in zc:
            pltpu.sync_copy(acc_sh.at[pl.ds(r0 + o, sz)],
                            buf_a.at[pl.ds(0, sz)])
            pltpu.sync_copy(buf_a.at[pl.ds(0, sz)],
                            out_hbm.at[cid, pl.ds(r0 + o, sz)])

    return sc_spmm


_BN_S = 1.0 / (1.0 + EPS_BN) ** 0.5


def _ln_body(a, w, b):
    mu = jnp.mean(a, axis=-1, keepdims=True)
    var = jnp.mean((a - mu) ** 2, axis=-1, keepdims=True)
    return (a - mu) * lax.rsqrt(var + EPS_LN) * w[None, :] + b[None, :]


def _tc_init_body(x_ref, gfw_ref, gfb_ref, gbw_ref, gbb_ref, tfw_ref, tfb_ref,
                  tlw_ref, tlb_ref, h0_ref, z0_ref):
    x = x_ref[...]
    h = jnp.dot(x, gfw_ref[...], preferred_element_type=jnp.float32)
    h = h + gfb_ref[...][None, :]
    h = h * (_BN_S * gbw_ref[...])[None, :] + gbb_ref[...][None, :]
    h0_ref[...] = jnp.maximum(h, 0.0)
    z = jnp.dot(x, tfw_ref[...], preferred_element_type=jnp.float32)
    z = z + tfb_ref[...][None, :]
    z = _ln_body(z, tlw_ref[...], tlb_ref[...])
    z0_ref[...] = jnp.maximum(z, 0.0)


def _tc_trans_body(z0_ref, qkv_ref, lnw_ref, lnb_ref, x1_ref, *, layers, n):
    z = z0_ref[...]
    last = z
    fn = jnp.float32(n)
    for l in range(layers):
        q = jnp.dot(z, qkv_ref[l, 0], preferred_element_type=jnp.float32)
        k = jnp.dot(z, qkv_ref[l, 1], preferred_element_type=jnp.float32)
        v = jnp.dot(z, qkv_ref[l, 2], preferred_element_type=jnp.float32)
        inv_qk = lax.rsqrt(jnp.sum(q * q) * jnp.sum(k * k))
        kvs = lax.dot_general(k, v, (((0,), (0,)), ((), ())),
                              preferred_element_type=jnp.float32)
        ks = jnp.sum(k, axis=0)
        num = jnp.dot(q, kvs, preferred_element_type=jnp.float32) * inv_qk \
            + fn * v
        den = jnp.sum(q * ks[None, :], axis=1, keepdims=True) * inv_qk + fn
        a = (num / den + last) * 0.5
        a = _ln_body(a, lnw_ref[l + 1], lnb_ref[l + 1])
        z = jnp.maximum(a, 0.0)
        last = z
    x1_ref[...] = z


def _tc_dinv_hw_body(degp_ref, h0_ref, w1_ref, dinv_ref, hws_ref, *, D):
    n = h0_ref.shape[0]
    d = degp_ref[0, 0:n, 0:1] + degp_ref[1, 0:n, 0:1] + 1.0
    dinv = jnp.broadcast_to(lax.rsqrt(d), (n, D))
    dinv_ref[...] = dinv
    hw = jnp.dot(h0_ref[...], w1_ref[...], preferred_element_type=jnp.float32)
    hws_ref[...] = dinv * hw


def _tc_gcn_body(p_ref, hws_ref, h_ref, dinv_ref, bnw_ref, bnb_ref, cb_ref,
                 wn_ref, hn_ref, hwsn_ref):
    dinv = dinv_ref[...]
    n = dinv.shape[0]
    agg = dinv * (p_ref[0, 0:n] + p_ref[1, 0:n] + hws_ref[...]) \
        + cb_ref[...][None, :]
    c = jnp.maximum(agg * (_BN_S * bnw_ref[...])[None, :]
                    + bnb_ref[...][None, :], 0.0)
    hn = c + h_ref[...]
    hn_ref[...] = hn
    if wn_ref is not None:
        hw = jnp.dot(hn, wn_ref[...], preferred_element_type=jnp.float32)
        hwsn_ref[...] = dinv * hw


def _tc_gcn_last_body(p_ref, hws_ref, h_ref, dinv_ref, bnw_ref, bnb_ref,
                      cb_ref, hn_ref):
    _tc_gcn_body(p_ref, hws_ref, h_ref, dinv_ref, bnw_ref, bnb_ref, cb_ref,
                 None, hn_ref, None)


def _tc_final_body(h_ref, x1_ref, fcw_ref, fcb_ref, out_ref):
    o = 0.5 * h_ref[...] + 0.5 * x1_ref[...]
    t = jnp.dot(o, fcw_ref[...], preferred_element_type=jnp.float32)
    t = t + fcb_ref[...][None, :]
    m = jnp.max(t, axis=-1, keepdims=True)
    e = jnp.exp(t - m)
    s = jnp.sum(e, axis=-1, keepdims=True)
    out_ref[...] = t - m - jnp.log(s)


def _tc_call(body, out_shapes, *args, **static):
    if static:
        body = functools.partial(body, **static)
    return pl.pallas_call(body, out_shape=out_shapes)(*args)


def kernel(x, edge_index, batch, g_fc_w, g_fc_b, g_bn_w, g_bn_b, g_conv_w,
           g_conv_b, t_fc_w, t_fc_b, t_ln_w, t_ln_b, t_qkv_w, fc_w, fc_b):
    N, D_IN = x.shape
    E = edge_index.shape[1]
    HID = g_fc_w.shape[1]
    OUT = fc_w.shape[1]
    gnn_layers = g_conv_w.shape[0]
    trans_layers = t_qkv_w.shape[0]

    rps = _rows_per_subcore(N)
    NP = rps * _NS
    align = _CHUNK * 8 * _NW
    E_p = -(-E // align) * align
    npad = E_p - E
    if npad:
        pad_src = jnp.arange(npad, dtype=jnp.int32) % N
        pad_dst = N + (jnp.arange(npad, dtype=jnp.int32) % (NP - N))
        src_f = jnp.concatenate([edge_index[0], pad_src])
        dst_f = jnp.concatenate([edge_index[1], pad_dst])
    else:
        src_f = edge_index[0]
        dst_f = edge_index[1]
    src2 = src_f.reshape(E_p // _CHUNK, _CHUNK)
    dst2 = dst_f.reshape(E_p // _CHUNK, _CHUNK)
    f32 = jnp.float32

    zeros_d = jnp.zeros((_CHUNK, HID), f32)
    ones_d = jnp.ones((_CHUNK, HID), f32)

    spmm = _make_sc_spmm(N, E_p, HID)

    degp = _make_sc_spmm(N, E_p, HID, gather=False)(ones_d, src2, dst2, zeros_d)

    h0, z0 = _tc_call(
        _tc_init_body,
        (jax.ShapeDtypeStruct((N, HID), f32),) * 2,
        x, g_fc_w, g_fc_b, g_bn_w[0], g_bn_b[0], t_fc_w, t_fc_b,
        t_ln_w[0], t_ln_b[0])

    x1 = _tc_call(
        _tc_trans_body,
        jax.ShapeDtypeStruct((N, HID), f32),
        z0, t_qkv_w, t_ln_w, t_ln_b, layers=trans_layers, n=N)

    dinv, hws = _tc_call(
        _tc_dinv_hw_body,
        (jax.ShapeDtypeStruct((N, HID), f32),) * 2,
        degp, h0, g_conv_w[0], D=HID)

    h = h0
    for l in range(gnn_layers):
        p = spmm(hws, src2, dst2, zeros_d)
        if l + 1 < gnn_layers:
            h, hws = _tc_call(
                _tc_gcn_body,
                (jax.ShapeDtypeStruct((N, HID), f32),) * 2,
                p, hws, h, dinv, g_bn_w[l + 1], g_bn_b[l + 1], g_conv_b[l],
                g_conv_w[l + 1])
        else:
            h = _tc_call(
                _tc_gcn_last_body,
                jax.ShapeDtypeStruct((N, HID), f32),
                p, hws, h, dinv, g_bn_w[l + 1], g_bn_b[l + 1], g_conv_b[l])

    return _tc_call(
        _tc_final_body,
        jax.ShapeDtypeStruct((N, OUT), f32),
        h, x1, fc_w, fc_b)


# speedup vs baseline: 3.4625x; 1.0217x over previous
import functools

import jax
import jax.numpy as jnp
from jax import lax
from jax.experimental import pallas as pl
from jax.experimental.pallas import tpu as pltpu
from jax.experimental.pallas import tpu_sc as plsc

EPS_BN = 1e-5
EPS_LN = 1e-5

try:
    _info = plsc.get_sparse_core_info()
    _NC, _NS = _info.num_cores, _info.num_subcores
except Exception:
    _NC, _NS = 2, 16
_NW = _NC * _NS

_CHUNK = 128


def _chunks(total, step):
    out = []
    o = 0
    while o < total:
        out.append((o, min(step, total - o)))
        o += step
    return out


def _rows_per_subcore(N):
    return ((N + _NS * 8 - 1) // (_NS * 8)) * 8


@functools.lru_cache(maxsize=None)
def _make_sc_spmm(N, E, D, gather=True):
    nch = E // _CHUNK
    assert E % _CHUNK == 0 and nch % (8 * _NW) == 0
    nw_f = nch // _NW
    rps = _rows_per_subcore(N)
    NP = rps * _NS
    zc = _chunks(rps, _CHUNK)
    budget = (2097151 - NP * D) // _NS
    PH = nw_f
    while 2 * PH * _CHUNK + 2 * _CHUNK * D > budget:
        PH = (PH + 1) // 2
    while nw_f % PH:
        PH -= 1
    nphase = nw_f // PH
    pairs, odd = divmod(PH, 2)
    mesh = plsc.VectorSubcoreMesh(core_axis_name="c", subcore_axis_name="s")

    @functools.partial(
        pl.kernel,
        out_type=jax.ShapeDtypeStruct((_NC, NP, D), jnp.float32),
        mesh=mesh,
        scratch_types=[
            pltpu.VMEM((PH, _CHUNK), jnp.int32),
            pltpu.VMEM((PH, _CHUNK), jnp.int32),
            pltpu.VMEM((_CHUNK, D), jnp.float32),
            pltpu.VMEM((_CHUNK, D), jnp.float32),
            pltpu.VMEM_SHARED((NP, D), jnp.float32),
            pltpu.SemaphoreType.DMA,
            pltpu.SemaphoreType.DMA,
        ],
    )
    def sc_spmm(rows_hbm, src_hbm, dst_hbm, zeros_hbm, out_hbm, sidx, didx,
                buf_a, buf_b, acc_sh, sem_a, sem_b):
        cid = lax.axis_index("c")
        sid = lax.axis_index("s")
        wid = sid * _NC + cid
        r0 = sid * rps
        pltpu.sync_copy(zeros_hbm, buf_a)
        for o, sz in zc:
            pltpu.sync_copy(buf_a.at[pl.ds(0, sz)],
                            acc_sh.at[pl.ds(r0 + o, sz)])
        cbase = wid * nw_f
        if not gather:
            pltpu.sync_copy(rows_hbm, buf_a)
        plsc.subcore_barrier()

        if gather:

            def phase(p, carry):
                pb = cbase + p * PH
                pltpu.sync_copy(src_hbm.at[pl.ds(pb, PH)], sidx)
                pltpu.sync_copy(dst_hbm.at[pl.ds(pb, PH)], didx)
                pltpu.async_copy(rows_hbm.at[sidx.at[0]], buf_a, sem_a)

                def body(i, carry):
                    c0 = 2 * i
                    pltpu.async_copy(rows_hbm.at[sidx.at[c0 + 1]], buf_b,
                                     sem_b)
                    pltpu.make_async_copy(rows_hbm.at[sidx.at[c0]], buf_a,
                                          sem_a).wait()
                    pltpu.sync_copy(buf_a, acc_sh.at[didx.at[c0]], add=True)
                    cn = jnp.minimum(c0 + 2, PH - 1)
                    pltpu.async_copy(rows_hbm.at[sidx.at[cn]], buf_a, sem_a)
                    pltpu.make_async_copy(rows_hbm.at[sidx.at[c0 + 1]],
                                          buf_b, sem_b).wait()
                    pltpu.sync_copy(buf_b, acc_sh.at[didx.at[c0 + 1]],
                                    add=True)
                    return carry

                lax.fori_loop(0, pairs, body, 0)
                pltpu.make_async_copy(rows_hbm.at[sidx.at[PH - 1]], buf_a,
                                      sem_a).wait()
                if odd:
                    pltpu.sync_copy(buf_a, acc_sh.at[didx.at[PH - 1]],
                                    add=True)
                return carry

            lax.fori_loop(0, nphase, phase, 0)
        else:

            def phase(p, carry):
                pb = cbase + p * PH
                pltpu.sync_copy(dst_hbm.at[pl.ds(pb, PH)], didx)

                def body(i, carry):
                    pltpu.sync_copy(buf_a, acc_sh.at[didx.at[i]], add=True)
                    return carry

                lax.fori_loop(0, PH, body, 0)
                return carry

            lax.fori_loop(0, nphase, phase, 0)

        plsc.subcore_barrier()
        for o, sz in zc:
            pltpu.sync_copy(acc_sh.at[pl.ds(r0 + o, sz)],
                            buf_a.at[pl.ds(0, sz)])
            pltpu.sync_copy(buf_a.at[pl.ds(0, sz)],
                            out_hbm.at[cid, pl.ds(r0 + o, sz)])

    return sc_spmm


_BN_S = 1.0 / (1.0 + EPS_BN) ** 0.5


def _ln_body(a, w, b):
    mu = jnp.mean(a, axis=-1, keepdims=True)
    var = jnp.mean((a - mu) ** 2, axis=-1, keepdims=True)
    return (a - mu) * lax.rsqrt(var + EPS_LN) * w[None, :] + b[None, :]


def _tc_init_trans_body(x_ref, gfw_ref, gfb_ref, gbw_ref, gbb_ref, tfw_ref,
                        tfb_ref, qkv_ref, lnw_ref, lnb_ref, h0_ref, x1_ref,
                        *, layers, n):
    x = x_ref[...]
    h = jnp.dot(x, gfw_ref[...], preferred_element_type=jnp.float32)
    h = h + gfb_ref[...][None, :]
    h = h * (_BN_S * gbw_ref[...])[None, :] + gbb_ref[...][None, :]
    h0_ref[...] = jnp.maximum(h, 0.0)
    z = jnp.dot(x, tfw_ref[...], preferred_element_type=jnp.float32)
    z = z + tfb_ref[...][None, :]
    z = _ln_body(z, lnw_ref[0], lnb_ref[0])
    z = jnp.maximum(z, 0.0)
    last = z
    fn = jnp.float32(n)
    for l in range(layers):
        q = jnp.dot(z, qkv_ref[l, 0], preferred_element_type=jnp.float32)
        k = jnp.dot(z, qkv_ref[l, 1], preferred_element_type=jnp.float32)
        v = jnp.dot(z, qkv_ref[l, 2], preferred_element_type=jnp.float32)
        inv_qk = lax.rsqrt(jnp.sum(q * q) * jnp.sum(k * k))
        kvs = lax.dot_general(k, v, (((0,), (0,)), ((), ())),
                              preferred_element_type=jnp.float32)
        ks = jnp.sum(k, axis=0)
        num = jnp.dot(q, kvs, preferred_element_type=jnp.float32) * inv_qk \
            + fn * v
        den = jnp.sum(q * ks[None, :], axis=1, keepdims=True) * inv_qk + fn
        a = (num / den + last) * 0.5
        a = _ln_body(a, lnw_ref[l + 1], lnb_ref[l + 1])
        z = jnp.maximum(a, 0.0)
        last = z
    x1_ref[...] = z


def _tc_dinv_hw_body(degp_ref, h0_ref, w1_ref, dinv_ref, hws_ref, *, D):
    n = h0_ref.shape[0]
    d = degp_ref[0, 0:n, 0:1] + degp_ref[1, 0:n, 0:1] + 1.0
    dinv = jnp.broadcast_to(lax.rsqrt(d), (n, D))
    dinv_ref[...] = dinv
    hw = jnp.dot(h0_ref[...], w1_ref[...], preferred_element_type=jnp.float32)
    hws_ref[...] = dinv * hw


def _tc_gcn_body(p_ref, hws_ref, h_ref, dinv_ref, bnw_ref, bnb_ref, cb_ref,
                 wn_ref, hn_ref, hwsn_ref):
    dinv = dinv_ref[...]
    n = dinv.shape[0]
    agg = dinv * (p_ref[0, 0:n] + p_ref[1, 0:n] + hws_ref[...]) \
        + cb_ref[...][None, :]
    c = jnp.maximum(agg * (_BN_S * bnw_ref[...])[None, :]
                    + bnb_ref[...][None, :], 0.0)
    hn = c + h_ref[...]
    hn_ref[...] = hn
    if wn_ref is not None:
        hw = jnp.dot(hn, wn_ref[...], preferred_element_type=jnp.float32)
        hwsn_ref[...] = dinv * hw


def _tc_gcn_final_body(p_ref, hws_ref, h_ref, dinv_ref, bnw_ref, bnb_ref,
                       cb_ref, x1_ref, fcw_ref, fcb_ref, out_ref):
    dinv = dinv_ref[...]
    n = dinv.shape[0]
    agg = dinv * (p_ref[0, 0:n] + p_ref[1, 0:n] + hws_ref[...]) \
        + cb_ref[...][None, :]
    c = jnp.maximum(agg * (_BN_S * bnw_ref[...])[None, :]
                    + bnb_ref[...][None, :], 0.0)
    hn = c + h_ref[...]
    o = 0.5 * hn + 0.5 * x1_ref[...]
    t = jnp.dot(o, fcw_ref[...], preferred_element_type=jnp.float32)
    t = t + fcb_ref[...][None, :]
    m = jnp.max(t, axis=-1, keepdims=True)
    e = jnp.exp(t - m)
    s = jnp.sum(e, axis=-1, keepdims=True)
    out_ref[...] = t - m - jnp.log(s)


def _tc_call(body, out_shapes, *args, **static):
    if static:
        body = functools.partial(body, **static)
    return pl.pallas_call(body, out_shape=out_shapes)(*args)


def kernel(x, edge_index, batch, g_fc_w, g_fc_b, g_bn_w, g_bn_b, g_conv_w,
           g_conv_b, t_fc_w, t_fc_b, t_ln_w, t_ln_b, t_qkv_w, fc_w, fc_b):
    N, D_IN = x.shape
    E = edge_index.shape[1]
    HID = g_fc_w.shape[1]
    OUT = fc_w.shape[1]
    gnn_layers = g_conv_w.shape[0]
    trans_layers = t_qkv_w.shape[0]

    rps = _rows_per_subcore(N)
    NP = rps * _NS
    align = _CHUNK * 8 * _NW
    E_p = -(-E // align) * align
    npad = E_p - E
    if npad:
        pad_src = jnp.arange(npad, dtype=jnp.int32) % N
        pad_dst = N + (jnp.arange(npad, dtype=jnp.int32) % (NP - N))
        src_f = jnp.concatenate([edge_index[0], pad_src])
        dst_f = jnp.concatenate([edge_index[1], pad_dst])
    else:
        src_f = edge_index[0]
        dst_f = edge_index[1]
    src2 = src_f.reshape(E_p // _CHUNK, _CHUNK)
    dst2 = dst_f.reshape(E_p // _CHUNK, _CHUNK)
    f32 = jnp.float32

    zeros_d = jnp.zeros((_CHUNK, HID), f32)
    ones_d = jnp.ones((_CHUNK, HID), f32)

    spmm = _make_sc_spmm(N, E_p, HID)

    degp = _make_sc_spmm(N, E_p, HID, gather=False)(ones_d, src2, dst2, zeros_d)

    h0, x1 = _tc_call(
        _tc_init_trans_body,
        (jax.ShapeDtypeStruct((N, HID), f32),) * 2,
        x, g_fc_w, g_fc_b, g_bn_w[0], g_bn_b[0], t_fc_w, t_fc_b,
        t_qkv_w, t_ln_w, t_ln_b, layers=trans_layers, n=N)

    dinv, hws = _tc_call(
        _tc_dinv_hw_body,
        (jax.ShapeDtypeStruct((N, HID), f32),) * 2,
        degp, h0, g_conv_w[0], D=HID)

    h = h0
    for l in range(gnn_layers):
        p = spmm(hws, src2, dst2, zeros_d)
        if l + 1 < gnn_layers:
            h, hws = _tc_call(
                _tc_gcn_body,
                (jax.ShapeDtypeStruct((N, HID), f32),) * 2,
                p, hws, h, dinv, g_bn_w[l + 1], g_bn_b[l + 1], g_conv_b[l],
                g_conv_w[l + 1])
        else:
            h = _tc_call(
                _tc_gcn_final_body,
                jax.ShapeDtypeStruct((N, OUT), f32),
                p, hws, h, dinv, g_bn_w[l + 1], g_bn_b[l + 1], g_conv_b[l],
                x1, fc_w, fc_b)
    return h
